# Initial kernel scaffold; baseline (speedup 1.0000x reference)
import jax
import jax.numpy as jnp
from jax.experimental import pallas as pl


def kernel(x, edge_index, batch, W1, b1, W2, b2, attW, attb, projW, projb, c1W, c1b, c2W, c2b):
    raise NotImplementedError("write your pallas kernel here")


# speedup vs baseline: 11.9271x; 11.9271x over previous
import functools

import jax
import jax.numpy as jnp
from jax import lax
from jax.experimental import pallas as pl
from jax.experimental.pallas import tpu as pltpu
from jax.experimental.pallas import tpu_sc as plsc

N = 10000
E = 320000
D = 128
H = 128
G = 256
EMB = 300
C = 10

NC = 2
NS = 16
NW = NC * NS
EPT = E // NW
CHUNK = 80
NCHUNK = EPT // CHUNK
NP = 10240
RPT = NP // NS

_mesh = plsc.VectorSubcoreMesh(core_axis_name="c", subcore_axis_name="s")

_HIGH = lax.Precision.HIGHEST


@functools.partial(
    pl.kernel,
    out_type=jax.ShapeDtypeStruct((NC, NP, D), jnp.float32),
    mesh=_mesh,
    scratch_types=[
        pltpu.VMEM((CHUNK,), jnp.int32),
        pltpu.VMEM((CHUNK, D), jnp.float32),
        pltpu.VMEM_SHARED((NP, D), jnp.float32),
    ],
)
def _sc_deg(zeros_hbm, ones_hbm, dst_hbm, out_hbm, dst_v, ones_v, acc_sh):
    c = lax.axis_index("c")
    s = lax.axis_index("s")
    wid = s * NC + c
    row0 = s * RPT
    pltpu.sync_copy(zeros_hbm.at[pl.ds(row0, RPT)], acc_sh.at[pl.ds(row0, RPT)])
    pltpu.sync_copy(ones_hbm, ones_v)
    plsc.subcore_barrier()

    def body(i, carry):
        base = wid * EPT + i * CHUNK
        pltpu.sync_copy(dst_hbm.at[pl.ds(base, CHUNK)], dst_v)
        pltpu.sync_copy(ones_v, acc_sh.at[dst_v], add=True)
        return carry

    lax.fori_loop(0, NCHUNK, body, 0)
    plsc.subcore_barrier()
    pltpu.sync_copy(acc_sh.at[pl.ds(row0, RPT)], out_hbm.at[c, pl.ds(row0, RPT)])


@functools.partial(
    pl.kernel,
    out_type=jax.ShapeDtypeStruct((NC, NP, D), jnp.float32),
    mesh=_mesh,
    scratch_types=[
        pltpu.VMEM((CHUNK,), jnp.int32),
        pltpu.VMEM((CHUNK,), jnp.int32),
        pltpu.VMEM((CHUNK, D), jnp.float32),
        pltpu.VMEM_SHARED((NP, D), jnp.float32),
        pltpu.SemaphoreType.DMA,
    ],
)
def _sc_conv(zeros_hbm, hs_hbm, src_hbm, dst_hbm, out_hbm,
             src_v, dst_v, rows_v, acc_sh, sem):
    c = lax.axis_index("c")
    s = lax.axis_index("s")
    wid = s * NC + c
    row0 = s * RPT
    pltpu.sync_copy(zeros_hbm.at[pl.ds(row0, RPT)], acc_sh.at[pl.ds(row0, RPT)])
    plsc.subcore_barrier()

    def body(i, carry):
        base = wid * EPT + i * CHUNK
        pltpu.sync_copy(src_hbm.at[pl.ds(base, CHUNK)], src_v)
        pltpu.async_copy(hs_hbm.at[src_v], rows_v, sem).wait()
        pltpu.sync_copy(dst_hbm.at[pl.ds(base, CHUNK)], dst_v)
        pltpu.sync_copy(rows_v, acc_sh.at[dst_v], add=True)
        return carry

    lax.fori_loop(0, NCHUNK, body, 0)
    plsc.subcore_barrier()
    pltpu.sync_copy(acc_sh.at[pl.ds(row0, RPT)], out_hbm.at[c, pl.ds(row0, RPT)])


def _tc_pre_body(parts_ref, x_ref, w1_ref, hs_ref, dis_ref):
    cnt = parts_ref[0, :, 0:1] + parts_ref[1, :, 0:1]
    dis = lax.rsqrt(cnt + 1.0)
    h = jnp.dot(x_ref[...], w1_ref[...],
                preferred_element_type=jnp.float32, precision=_HIGH)
    hs_ref[...] = h * dis
    dis_ref[...] = dis


_tc_pre = pl.pallas_call(
    _tc_pre_body,
    out_shape=[
        jax.ShapeDtypeStruct((N, D), jnp.float32),
        jax.ShapeDtypeStruct((N, 1), jnp.float32),
    ],
)


def _tc_mid_body(acc_ref, hs_ref, dis_ref, b_ref, w2_ref, hs2_ref):
    tot = acc_ref[0] + acc_ref[1] + hs_ref[...]
    h1 = jnp.maximum(dis_ref[...] * tot + b_ref[...], 0.0)
    hs2_ref[...] = jnp.dot(h1, w2_ref[...],
                           preferred_element_type=jnp.float32,
                           precision=_HIGH) * dis_ref[...]


_tc_mid = pl.pallas_call(
    _tc_mid_body,
    out_shape=[jax.ShapeDtypeStruct((N, H), jnp.float32)],
)


def _tc_post_body(acc_ref, hs2_ref, dis_ref, b2_ref, attw_ref, attb_ref,
                  batch_ref, projw_ref, projb_ref, c1w_ref, c1b_ref,
                  c2w_ref, c2b_ref, out_ref):
    tot = acc_ref[0] + acc_ref[1] + hs2_ref[...]
    h2 = jnp.maximum(dis_ref[...] * tot + b2_ref[...], 0.0)
    z = jnp.dot(h2, attw_ref[...], preferred_element_type=jnp.float32,
                precision=_HIGH) + attb_ref[...]
    att = jnp.where(z > 0, z, 0.01 * z)
    b = batch_ref[...]
    gid = lax.broadcasted_iota(jnp.int32, (N, G), 1)
    mask = b == gid
    onehot = mask.astype(jnp.float32)
    segmax = jnp.max(jnp.where(mask, att, -1e30), axis=0, keepdims=True)
    maxn = lax.dot_general(onehot, segmax, (((1,), (1,)), ((), ())),
                           preferred_element_type=jnp.float32,
                           precision=_HIGH)
    e = jnp.exp(att - maxn)
    denom = lax.dot_general(onehot, e, (((0,), (0,)), ((), ())),
                            preferred_element_type=jnp.float32,
                            precision=_HIGH)
    num = lax.dot_general(onehot, e * h2, (((0,), (0,)), ((), ())),
                          preferred_element_type=jnp.float32,
                          precision=_HIGH)
    g = num / (denom + 1e-16)
    p = jnp.dot(g, projw_ref[...], preferred_element_type=jnp.float32,
                precision=_HIGH) + projb_ref[...]
    q = jnp.maximum(jnp.dot(p, c1w_ref[...],
                            preferred_element_type=jnp.float32,
                            precision=_HIGH) + c1b_ref[...], 0.0)
    out_ref[...] = jnp.dot(q, c2w_ref[...],
                           preferred_element_type=jnp.float32,
                           precision=_HIGH) + c2b_ref[...]


_tc_post = pl.pallas_call(
    _tc_post_body,
    out_shape=[jax.ShapeDtypeStruct((G, C), jnp.float32)],
)


def kernel(x, edge_index, batch, W1, b1, W2, b2, attW, attb,
           projW, projb, c1W, c1b, c2W, c2b):
    src = edge_index[0]
    dst = edge_index[1]
    zeros_nd = jnp.zeros((NP, D), jnp.float32)
    ones_cd = jnp.ones((CHUNK, D), jnp.float32)

    deg_parts = _sc_deg(zeros_nd, ones_cd, dst)[:, :N, :]
    hs1, dis = _tc_pre(deg_parts, x, W1)
    acc1 = _sc_conv(zeros_nd, hs1, src, dst)[:, :N, :]
    (hs2,) = _tc_mid(acc1, hs1, dis, b1.reshape(1, H), W2)
    acc2 = _sc_conv(zeros_nd, hs2, src, dst)[:, :N, :]
    (out,) = _tc_post(acc2, hs2, dis, b2.reshape(1, H), attW,
                      attb.reshape(1, 1), batch.reshape(N, 1), projW,
                      projb.reshape(1, EMB), c1W, c1b.reshape(1, 128),
                      c2W, c2b.reshape(1, C))
    return out
